# baseline (speedup 1.0000x reference)
import jax
import jax.numpy as jnp
from jax import lax
from jax.experimental import pallas as pl
from jax.experimental.pallas import tpu as pltpu
from jax.experimental.pallas import tpu_sc as plsc

N = 10000
E = 320000
D = 128
R = 4
NC = 2
NS = 16
CW = 32
NCH = D // CW
STREAM = 400
SEGROWS = R * N
EPC = E // (NC * NS)
EPA = E // NS

_MESH = dict(core_axis_name="c", subcore_axis_name="s", num_cores=NC,
             num_subcores=NS)


def _i32(v):
    return jnp.int32(v)


def _fori(n, body):
    lax.fori_loop(jnp.int32(0), jnp.int32(n), body, jnp.int32(0))


def _zero_fill(ref, nrows, width):
    def body(j, carry):
        for k in range(width // 16):
            ref[j, pl.ds(k * 16, 16)] = jnp.zeros((16,), jnp.float32)
        return carry
    _fori(nrows, body)


CB = 2000


def _count_body(dst1, et1, seg1, cnt_a, cnt_b, cntsh, dstv, etv, segv, segc,
                ones, zb, sem):
    core = lax.axis_index("c")
    sub = lax.axis_index("s")
    _zero_fill(zb, 500, 16)
    def ones_body(j, carry):
        ones[j, :] = jnp.ones((16,), jnp.float32)
        return carry
    _fori(CB, ones_body)
    r0 = sub * _i32(SEGROWS // NS)
    for k in range(SEGROWS // NS // 500):
        pltpu.sync_copy(zb, cntsh.at[pl.ds(r0 + k * 500, 500)])
    plsc.subcore_barrier()

    wid = core * NS + sub
    n_outer = EPC // CB

    def outer(i, carry):
        eb = wid * _i32(EPC) + i * _i32(CB)
        pltpu.sync_copy(dst1.at[pl.ds(eb, CB)], dstv)
        pltpu.sync_copy(et1.at[pl.ds(eb, CB)], etv)

        def comp(j, c2):
            sl = pl.ds(j * 16, 16)
            dv = dstv[sl]
            ev = etv[sl]
            segv[sl] = ev * _i32(N) + dv
            segc[sl] = dv * _i32(R) + ev
            return c2
        _fori(CB // 16, comp)
        pltpu.sync_copy(segv, seg1.at[pl.ds(eb, CB)])
        pltpu.async_copy(ones, cntsh.at[segc], sem, add=True).wait()
        return carry
    _fori(n_outer, outer)
    plsc.subcore_barrier()

    stripe = SEGROWS // NS
    @pl.when(core == 0)
    def _():
        pltpu.sync_copy(cntsh.at[pl.ds(r0, stripe)],
                        cnt_a.at[pl.ds(r0, stripe)])
    @pl.when(core == 1)
    def _():
        pltpu.sync_copy(cntsh.at[pl.ds(r0, stripe)],
                        cnt_b.at[pl.ds(r0, stripe)])


@jax.jit
def _count_kernel(dst1, et1):
    mesh = plsc.VectorSubcoreMesh(**_MESH)
    return pl.kernel(
        _count_body,
        out_type=[
            jax.ShapeDtypeStruct((E,), jnp.int32),
            jax.ShapeDtypeStruct((SEGROWS, 16), jnp.float32),
            jax.ShapeDtypeStruct((SEGROWS, 16), jnp.float32),
        ],
        mesh=mesh,
        compiler_params=pltpu.CompilerParams(use_tc_tiling_on_sc=False),
        scratch_types=[
            pltpu.VMEM_SHARED((SEGROWS, 16), jnp.float32),
            pltpu.VMEM((CB,), jnp.int32),
            pltpu.VMEM((CB,), jnp.int32),
            pltpu.VMEM((CB,), jnp.int32),
            pltpu.VMEM((CB,), jnp.int32),
            pltpu.VMEM((CB, 16), jnp.float32),
            pltpu.VMEM((500, 16), jnp.float32),
            pltpu.SemaphoreType.DMA,
        ],
    )(dst1, et1)


def _agg_body(t0, t1, src1, seg1, a0, a1,
              accsh, sidx, segv, rows, zb, isem, gsem, ssem):
    core = lax.axis_index("c")
    sub = lax.axis_index("s")
    _zero_fill(zb, 250, CW)
    tables = [t0, t1]
    outs = [a0, a1]
    r0 = sub * _i32(SEGROWS // NS)
    stripe = SEGROWS // NS
    n_outer = EPA // STREAM

    def scatter_pass(table):
        idx_cps = {}

        def fire_idx(i):
            bb = _i32(i % 3)
            eb = sub * _i32(EPA) + _i32(i * STREAM)
            idx_cps[i] = (
                pltpu.async_copy(src1.at[pl.ds(eb, STREAM)], sidx.at[bb], isem),
                pltpu.async_copy(seg1.at[pl.ds(eb, STREAM)], segv.at[bb], isem),
            )

        fire_idx(0)
        pend = {}
        for i in range(n_outer):
            b3 = _i32(i % 3)
            b2 = _i32(i % 2)
            for cp in idx_cps.pop(i):
                cp.wait()
            if i >= 2:
                pend.pop(i - 2).wait()
            if i + 1 < n_outer:
                fire_idx(i + 1)
            gcp = pltpu.async_copy(table.at[sidx.at[b3]], rows.at[b2], gsem)
            gcp.wait()
            pend[i] = pltpu.async_copy(rows.at[b2], accsh.at[segv.at[b3]],
                                       ssem, add=True)
        for gi in sorted(pend):
            pend[gi].wait()

    for k in range(stripe // 250):
        pltpu.sync_copy(zb, accsh.at[pl.ds(r0 + k * 250, 250)])
    plsc.subcore_barrier()
    for ch in range(2):
        @pl.when(core == ch)
        def _(ch=ch):
            scatter_pass(tables[ch])
    plsc.subcore_barrier()
    plane = sub // _i32(N // (SEGROWS // NS))
    prow = (sub % _i32(N // (SEGROWS // NS))) * _i32(stripe)
    for ch in range(2):
        @pl.when(core == ch)
        def _(ch=ch):
            pltpu.sync_copy(accsh.at[pl.ds(r0, stripe)],
                            outs[ch].at[plane, pl.ds(prow, stripe)])


@jax.jit
def _agg_kernel(t0, t1, src1, seg1):
    mesh = plsc.VectorSubcoreMesh(**_MESH)
    out = jax.ShapeDtypeStruct((R, N, CW), jnp.float32)
    return pl.kernel(
        _agg_body,
        out_type=[out, out],
        mesh=mesh,
        compiler_params=pltpu.CompilerParams(use_tc_tiling_on_sc=False),
        scratch_types=[
            pltpu.VMEM_SHARED((SEGROWS, CW), jnp.float32),
            pltpu.VMEM((3, STREAM), jnp.int32),
            pltpu.VMEM((3, STREAM), jnp.int32),
            pltpu.VMEM((2, STREAM, CW), jnp.float32),
            pltpu.VMEM((250, CW), jnp.float32),
            pltpu.SemaphoreType.DMA,
            pltpu.SemaphoreType.DMA,
            pltpu.SemaphoreType.DMA,
        ],
    )(t0, t1, src1, seg1)


BN = 200


def _pack_f64_u32(out):
    xu = lax.bitcast_convert_type(out, jnp.uint32)
    sgn = xu & jnp.uint32(0x80000000)
    ex = (xu >> jnp.uint32(23)) & jnp.uint32(0xFF)
    mn = xu & jnp.uint32(0x7FFFFF)
    nz = ex > jnp.uint32(0)
    hi = jnp.where(nz,
                   sgn | ((ex + jnp.uint32(896)) << jnp.uint32(20))
                   | (mn >> jnp.uint32(3)),
                   jnp.uint32(0))
    lo = jnp.where(nz, (mn & jnp.uint32(7)) << jnp.uint32(29), jnp.uint32(0))
    inter = jnp.stack([lo, hi], axis=-1)
    return inter.reshape(out.shape[0], out.shape[1] * 2)


def _combine_body_hc(a0, a1, a2, a3, cnt, xr, wfull, wroot, b,
                     h, u64, hc0, hc1, hc2, hc3):
    _combine_core(a0, a1, a2, a3, cnt, xr, wfull, wroot, b,
                  h, u64, [hc0, hc1, hc2, hc3])


def _combine_body_last(a0, a1, a2, a3, cnt, xr, wfull, wroot, b, h, u64):
    _combine_core(a0, a1, a2, a3, cnt, xr, wfull, wroot, b, h, u64, None)


def _combine_core(a0, a1, a2, a3, cnt, xr, wfull, wroot, b, h, u64, hcs):
    cntarr = cnt[...]
    rec = 1.0 / jnp.maximum(cntarr[0] + cntarr[1], 1.0)
    acc = jnp.dot(xr[...], wroot[...], preferred_element_type=jnp.float32)
    arefs = [a0, a1, a2, a3]
    pieces = []
    for r in range(R):
        mr = jnp.concatenate([arefs[c][r] for c in range(NCH)], axis=1)
        pieces.append(mr * rec[:, r:r + 1])
    m = jnp.concatenate(pieces, axis=1)
    acc = acc + jnp.dot(m, wfull[...], preferred_element_type=jnp.float32)
    out = jnp.maximum(acc + b[...], 0.0)
    h[...] = out
    u64[...] = _pack_f64_u32(out)
    if hcs is not None:
        for c in range(NCH):
            hcs[c][...] = out[:, c * CW:(c + 1) * CW]


def _make_combine(emit_hc):
    body = _combine_body_hc if emit_hc else _combine_body_last
    z = lambda: jnp.int32(0)
    ablock = pl.BlockSpec((R, BN, CW), lambda i: (z(), i, z()))
    out_specs = [
        pl.BlockSpec((BN, D), lambda i: (i, z())),
        pl.BlockSpec((BN, 2 * D), lambda i: (i, z())),
    ]
    out_shape = [
        jax.ShapeDtypeStruct((N, D), jnp.float32),
        jax.ShapeDtypeStruct((N, 2 * D), jnp.uint32),
    ]
    if emit_hc:
        out_specs += [pl.BlockSpec((BN, CW), lambda i: (i, z()))] * NCH
        out_shape += [jax.ShapeDtypeStruct((N, CW), jnp.float32)] * NCH

    @jax.jit
    def run(a0, a1, a2, a3, cnt, xin, wfull, wroot, b):
        return pl.pallas_call(
            body,
            grid=(N // BN,),
            in_specs=[
                ablock, ablock, ablock, ablock,
                pl.BlockSpec((2, BN, R), lambda i: (z(), i, z())),
                pl.BlockSpec((BN, D), lambda i: (i, z())),
                pl.BlockSpec((R * D, D), lambda i: (z(), z())),
                pl.BlockSpec((D, D), lambda i: (z(), z())),
                pl.BlockSpec((1, D), lambda i: (z(), z())),
            ],
            out_specs=out_specs,
            out_shape=out_shape,
        )(a0, a1, a2, a3, cnt, xin, wfull, wroot, b)
    return run


_combine_hc = _make_combine(True)
_combine_last = _make_combine(False)


def _layer(tables, src1, seg1, cnt, xin, Wr, Wroot, b, last):
    a01 = _agg_kernel(tables[0], tables[1], src1, seg1)
    a23 = _agg_kernel(tables[2], tables[3], src1, seg1)
    a = list(a01) + list(a23)
    wfull = Wr.astype(jnp.float32).reshape(R * D, D)
    fn = _combine_last if last else _combine_hc
    res = fn(a[0], a[1], a[2], a[3], cnt, xin, wfull,
             Wroot.astype(jnp.float32),
             b.reshape(1, D).astype(jnp.float32))
    h, u64 = res[0], res[1]
    return h, u64, list(res[2:])


def kernel(x, edge_index, edge_attr, Wr1, Wroot1, b1, Wr2, Wroot2, b2):
    x = x.astype(jnp.float32)
    src1 = edge_index[0].astype(jnp.int32)
    dst1 = edge_index[1].astype(jnp.int32)
    et1 = edge_attr[:, 0].astype(jnp.int32)

    seg1, cnt_a, cnt_b = _count_kernel(dst1, et1)
    cnt = jnp.stack([cnt_a[:, 0].reshape(N, R), cnt_b[:, 0].reshape(N, R)])

    tables1 = [x[:, c * CW:(c + 1) * CW] for c in range(NCH)]
    h1, u1, tables2 = _layer(tables1, src1, seg1, cnt, x, Wr1, Wroot1, b1,
                             last=False)
    _, u2, _ = _layer(tables2, src1, seg1, cnt, h1, Wr2, Wroot2, b2,
                      last=True)
    packed = jnp.stack([u1, u2]).reshape(2, N, D, 2)
    return lax.bitcast_convert_type(packed, jnp.float64)


# speedup vs baseline: 2.5933x; 2.5933x over previous
import jax
import jax.numpy as jnp
from jax import lax
from jax.experimental import pallas as pl
from jax.experimental.pallas import tpu as pltpu
from jax.experimental.pallas import tpu_sc as plsc

N = 10000
E = 320000
D = 128
R = 4
NC = 2
NS = 16
CW = 32
NCH = D // CW
STREAM = 400
SEGROWS = R * N
EPC = E // (NC * NS)
EPA = E // NS

_MESH = dict(core_axis_name="c", subcore_axis_name="s", num_cores=NC,
             num_subcores=NS)


def _i32(v):
    return jnp.int32(v)


def _fori(n, body):
    lax.fori_loop(jnp.int32(0), jnp.int32(n), body, jnp.int32(0))


def _zero_fill(ref, nrows, width):
    def body(j, carry):
        for k in range(width // 16):
            ref[j, pl.ds(k * 16, 16)] = jnp.zeros((16,), jnp.float32)
        return carry
    _fori(nrows, body)


CB = 2000


def _count_body(dst1, et1, seg1, cnt_a, cnt_b, cntsh, dstv, etv, segv, segc,
                ones, zb, sem):
    core = lax.axis_index("c")
    sub = lax.axis_index("s")
    _zero_fill(zb, 500, 16)
    def ones_body(j, carry):
        ones[j, :] = jnp.ones((16,), jnp.float32)
        return carry
    _fori(CB, ones_body)
    r0 = sub * _i32(SEGROWS // NS)
    for k in range(SEGROWS // NS // 500):
        pltpu.sync_copy(zb, cntsh.at[pl.ds(r0 + k * 500, 500)])
    plsc.subcore_barrier()

    wid = core * NS + sub
    n_outer = EPC // CB

    def outer(i, carry):
        eb = wid * _i32(EPC) + i * _i32(CB)
        pltpu.sync_copy(dst1.at[pl.ds(eb, CB)], dstv)
        pltpu.sync_copy(et1.at[pl.ds(eb, CB)], etv)

        def comp(j, c2):
            sl = pl.ds(j * 16, 16)
            dv = dstv[sl]
            ev = etv[sl]
            segv[sl] = ev * _i32(N) + dv
            segc[sl] = dv * _i32(R) + ev
            return c2
        _fori(CB // 16, comp)
        pltpu.sync_copy(segv, seg1.at[pl.ds(eb, CB)])
        pltpu.async_copy(ones, cntsh.at[segc], sem, add=True).wait()
        return carry
    _fori(n_outer, outer)
    plsc.subcore_barrier()

    stripe = SEGROWS // NS
    @pl.when(core == 0)
    def _():
        pltpu.sync_copy(cntsh.at[pl.ds(r0, stripe)],
                        cnt_a.at[pl.ds(r0, stripe)])
    @pl.when(core == 1)
    def _():
        pltpu.sync_copy(cntsh.at[pl.ds(r0, stripe)],
                        cnt_b.at[pl.ds(r0, stripe)])


@jax.jit
def _count_kernel(dst1, et1):
    mesh = plsc.VectorSubcoreMesh(**_MESH)
    return pl.kernel(
        _count_body,
        out_type=[
            jax.ShapeDtypeStruct((E,), jnp.int32),
            jax.ShapeDtypeStruct((SEGROWS, 16), jnp.float32),
            jax.ShapeDtypeStruct((SEGROWS, 16), jnp.float32),
        ],
        mesh=mesh,
        compiler_params=pltpu.CompilerParams(use_tc_tiling_on_sc=False),
        scratch_types=[
            pltpu.VMEM_SHARED((SEGROWS, 16), jnp.float32),
            pltpu.VMEM((CB,), jnp.int32),
            pltpu.VMEM((CB,), jnp.int32),
            pltpu.VMEM((CB,), jnp.int32),
            pltpu.VMEM((CB,), jnp.int32),
            pltpu.VMEM((CB, 16), jnp.float32),
            pltpu.VMEM((500, 16), jnp.float32),
            pltpu.SemaphoreType.DMA,
        ],
    )(dst1, et1)


def _agg_body(t0, t1, src1, seg1, a0, a1,
              accsh, sidx, segv, rows, zb, isem, gsem, ssem):
    core = lax.axis_index("c")
    sub = lax.axis_index("s")
    _zero_fill(zb, 250, CW)
    tables = [t0, t1]
    outs = [a0, a1]
    r0 = sub * _i32(SEGROWS // NS)
    stripe = SEGROWS // NS
    n_outer = EPA // STREAM

    def scatter_pass(table):
        idx_cps = {}

        def fire_idx(i):
            bb = _i32(i % 3)
            eb = sub * _i32(EPA) + _i32(i * STREAM)
            idx_cps[i] = (
                pltpu.async_copy(src1.at[pl.ds(eb, STREAM)], sidx.at[bb], isem),
                pltpu.async_copy(seg1.at[pl.ds(eb, STREAM)], segv.at[bb], isem),
            )

        fire_idx(0)
        pend = {}
        for i in range(n_outer):
            b3 = _i32(i % 3)
            b2 = _i32(i % 2)
            for cp in idx_cps.pop(i):
                cp.wait()
            if i >= 2:
                pend.pop(i - 2).wait()
            if i + 1 < n_outer:
                fire_idx(i + 1)
            gcp = pltpu.async_copy(table.at[sidx.at[b3]], rows.at[b2], gsem)
            gcp.wait()
            pend[i] = pltpu.async_copy(rows.at[b2], accsh.at[segv.at[b3]],
                                       ssem, add=True)
        for gi in sorted(pend):
            pend[gi].wait()

    for k in range(stripe // 250):
        pltpu.sync_copy(zb, accsh.at[pl.ds(r0 + k * 250, 250)])
    plsc.subcore_barrier()
    for ch in range(2):
        @pl.when(core == ch)
        def _(ch=ch):
            scatter_pass(tables[ch])
    plsc.subcore_barrier()
    plane = sub // _i32(N // (SEGROWS // NS))
    prow = (sub % _i32(N // (SEGROWS // NS))) * _i32(stripe)
    for ch in range(2):
        @pl.when(core == ch)
        def _(ch=ch):
            pltpu.sync_copy(accsh.at[pl.ds(r0, stripe)],
                            outs[ch].at[plane, pl.ds(prow, stripe)])


@jax.jit
def _agg_kernel(t0, t1, src1, seg1):
    mesh = plsc.VectorSubcoreMesh(**_MESH)
    out = jax.ShapeDtypeStruct((R, N, CW), jnp.float32)
    return pl.kernel(
        _agg_body,
        out_type=[out, out],
        mesh=mesh,
        compiler_params=pltpu.CompilerParams(use_tc_tiling_on_sc=False),
        scratch_types=[
            pltpu.VMEM_SHARED((SEGROWS, CW), jnp.float32),
            pltpu.VMEM((3, STREAM), jnp.int32),
            pltpu.VMEM((3, STREAM), jnp.int32),
            pltpu.VMEM((2, STREAM, CW), jnp.float32),
            pltpu.VMEM((250, CW), jnp.float32),
            pltpu.SemaphoreType.DMA,
            pltpu.SemaphoreType.DMA,
            pltpu.SemaphoreType.DMA,
        ],
    )(t0, t1, src1, seg1)


BN = 200


def _combine_body_hc(a0, a1, a2, a3, cnt, xr, wfull, wroot, b,
                     h, hc0, hc1, hc2, hc3):
    _combine_core(a0, a1, a2, a3, cnt, xr, wfull, wroot, b,
                  h, [hc0, hc1, hc2, hc3])


def _combine_body_last(a0, a1, a2, a3, cnt, xr, wfull, wroot, b, h):
    _combine_core(a0, a1, a2, a3, cnt, xr, wfull, wroot, b, h, None)


def _combine_core(a0, a1, a2, a3, cnt, xr, wfull, wroot, b, h, hcs):
    cntarr = cnt[...]
    rec = 1.0 / jnp.maximum(cntarr[0] + cntarr[1], 1.0)
    acc = jnp.dot(xr[...], wroot[...], preferred_element_type=jnp.float32)
    arefs = [a0, a1, a2, a3]
    pieces = []
    for r in range(R):
        mr = jnp.concatenate([arefs[c][r] for c in range(NCH)], axis=1)
        pieces.append(mr * rec[:, r:r + 1])
    m = jnp.concatenate(pieces, axis=1)
    acc = acc + jnp.dot(m, wfull[...], preferred_element_type=jnp.float32)
    out = jnp.maximum(acc + b[...], 0.0)
    h[...] = out
    if hcs is not None:
        for c in range(NCH):
            hcs[c][...] = out[:, c * CW:(c + 1) * CW]


def _make_combine(emit_hc):
    body = _combine_body_hc if emit_hc else _combine_body_last
    z = lambda: jnp.int32(0)
    ablock = pl.BlockSpec((R, BN, CW), lambda i: (z(), i, z()))
    out_specs = [
        pl.BlockSpec((BN, D), lambda i: (i, z())),
    ]
    out_shape = [
        jax.ShapeDtypeStruct((N, D), jnp.float32),
    ]
    if emit_hc:
        out_specs += [pl.BlockSpec((BN, CW), lambda i: (i, z()))] * NCH
        out_shape += [jax.ShapeDtypeStruct((N, CW), jnp.float32)] * NCH

    @jax.jit
    def run(a0, a1, a2, a3, cnt, xin, wfull, wroot, b):
        return pl.pallas_call(
            body,
            grid=(N // BN,),
            in_specs=[
                ablock, ablock, ablock, ablock,
                pl.BlockSpec((2, BN, R), lambda i: (z(), i, z())),
                pl.BlockSpec((BN, D), lambda i: (i, z())),
                pl.BlockSpec((R * D, D), lambda i: (z(), z())),
                pl.BlockSpec((D, D), lambda i: (z(), z())),
                pl.BlockSpec((1, D), lambda i: (z(), z())),
            ],
            out_specs=out_specs,
            out_shape=out_shape,
        )(a0, a1, a2, a3, cnt, xin, wfull, wroot, b)
    return run


_combine_hc = _make_combine(True)
_combine_last = _make_combine(False)


def _layer(tables, src1, seg1, cnt, xin, Wr, Wroot, b, last):
    a01 = _agg_kernel(tables[0], tables[1], src1, seg1)
    a23 = _agg_kernel(tables[2], tables[3], src1, seg1)
    a = list(a01) + list(a23)
    wfull = Wr.astype(jnp.float32).reshape(R * D, D)
    fn = _combine_last if last else _combine_hc
    res = fn(a[0], a[1], a[2], a[3], cnt, xin, wfull,
             Wroot.astype(jnp.float32),
             b.reshape(1, D).astype(jnp.float32))
    if last:
        return res[0], []
    return res[0], list(res[1:])


def kernel(x, edge_index, edge_attr, Wr1, Wroot1, b1, Wr2, Wroot2, b2):
    x = x.astype(jnp.float32)
    src1 = edge_index[0].astype(jnp.int32)
    dst1 = edge_index[1].astype(jnp.int32)
    et1 = edge_attr[:, 0].astype(jnp.int32)

    seg1, cnt_a, cnt_b = _count_kernel(dst1, et1)
    cnt = jnp.stack([cnt_a[:, 0].reshape(N, R), cnt_b[:, 0].reshape(N, R)])

    tables1 = [x[:, c * CW:(c + 1) * CW] for c in range(NCH)]
    h1, tables2 = _layer(tables1, src1, seg1, cnt, x, Wr1, Wroot1, b1,
                         last=False)
    h2, _ = _layer(tables2, src1, seg1, cnt, h1, Wr2, Wroot2, b2,
                   last=True)
    return jnp.stack([h1, h2]).astype(jnp.float64)


# speedup vs baseline: 2.8133x; 1.0848x over previous
import jax
import jax.numpy as jnp
from jax import lax
from jax.experimental import pallas as pl
from jax.experimental.pallas import tpu as pltpu
from jax.experimental.pallas import tpu_sc as plsc

N = 10000
E = 320000
D = 128
R = 4
NC = 2
NS = 16
CW = 32
NCH = D // CW
STREAM = 400
SEGROWS = R * N
EPC = E // (NC * NS)
EPA = E // NS

_MESH = dict(core_axis_name="c", subcore_axis_name="s", num_cores=NC,
             num_subcores=NS)


def _i32(v):
    return jnp.int32(v)


def _fori(n, body):
    lax.fori_loop(jnp.int32(0), jnp.int32(n), body, jnp.int32(0))


def _zero_fill(ref, nrows, width):
    def body(j, carry):
        for k in range(width // 16):
            ref[j, pl.ds(k * 16, 16)] = jnp.zeros((16,), jnp.float32)
        return carry
    _fori(nrows, body)


CB = 2000


def _count_body(dst1, et1, seg1, cnt_a, cnt_b, cntsh, dstv, etv, segv, segc,
                ones, zb, sem):
    core = lax.axis_index("c")
    sub = lax.axis_index("s")
    _zero_fill(zb, 500, 16)
    def ones_body(j, carry):
        ones[j, :] = jnp.ones((16,), jnp.float32)
        return carry
    _fori(CB, ones_body)
    r0 = sub * _i32(SEGROWS // NS)
    for k in range(SEGROWS // NS // 500):
        pltpu.sync_copy(zb, cntsh.at[pl.ds(r0 + k * 500, 500)])
    plsc.subcore_barrier()

    wid = core * NS + sub
    n_outer = EPC // CB

    def outer(i, carry):
        eb = wid * _i32(EPC) + i * _i32(CB)
        pltpu.sync_copy(dst1.at[pl.ds(eb, CB)], dstv)
        pltpu.sync_copy(et1.at[pl.ds(eb, CB)], etv)

        def comp(j, c2):
            sl = pl.ds(j * 16, 16)
            dv = dstv[sl]
            ev = etv[sl]
            segv[sl] = ev * _i32(N) + dv
            segc[sl] = dv * _i32(R) + ev
            return c2
        _fori(CB // 16, comp)
        pltpu.sync_copy(segv, seg1.at[pl.ds(eb, CB)])
        pltpu.async_copy(ones, cntsh.at[segc], sem, add=True).wait()
        return carry
    _fori(n_outer, outer)
    plsc.subcore_barrier()

    stripe = SEGROWS // NS
    @pl.when(core == 0)
    def _():
        pltpu.sync_copy(cntsh.at[pl.ds(r0, stripe)],
                        cnt_a.at[pl.ds(r0, stripe)])
    @pl.when(core == 1)
    def _():
        pltpu.sync_copy(cntsh.at[pl.ds(r0, stripe)],
                        cnt_b.at[pl.ds(r0, stripe)])


@jax.jit
def _count_kernel(dst1, et1):
    mesh = plsc.VectorSubcoreMesh(**_MESH)
    return pl.kernel(
        _count_body,
        out_type=[
            jax.ShapeDtypeStruct((E,), jnp.int32),
            jax.ShapeDtypeStruct((SEGROWS, 16), jnp.float32),
            jax.ShapeDtypeStruct((SEGROWS, 16), jnp.float32),
        ],
        mesh=mesh,
        compiler_params=pltpu.CompilerParams(use_tc_tiling_on_sc=False),
        scratch_types=[
            pltpu.VMEM_SHARED((SEGROWS, 16), jnp.float32),
            pltpu.VMEM((CB,), jnp.int32),
            pltpu.VMEM((CB,), jnp.int32),
            pltpu.VMEM((CB,), jnp.int32),
            pltpu.VMEM((CB,), jnp.int32),
            pltpu.VMEM((CB, 16), jnp.float32),
            pltpu.VMEM((500, 16), jnp.float32),
            pltpu.SemaphoreType.DMA,
        ],
    )(dst1, et1)


def _agg_body(t0, t1, src1, seg1, a0, a1,
              accsh, sidx, segv, rows, zb, isem, gsem, ssem):
    core = lax.axis_index("c")
    sub = lax.axis_index("s")
    def zb_body(j, carry):
        zb[j, :] = jnp.zeros((32,), jnp.bfloat16)
        return carry
    _fori(250, zb_body)
    tables = [t0, t1]
    outs = [a0, a1]
    r0 = sub * _i32(SEGROWS // NS)
    stripe = SEGROWS // NS
    n_outer = EPA // STREAM

    def scatter_pass(table):
        idx_cps = {}

        def fire_idx(i):
            bb = _i32(i % 3)
            eb = sub * _i32(EPA) + _i32(i * STREAM)
            idx_cps[i] = (
                pltpu.async_copy(src1.at[pl.ds(eb, STREAM)], sidx.at[bb], isem),
                pltpu.async_copy(seg1.at[pl.ds(eb, STREAM)], segv.at[bb], isem),
            )

        fire_idx(0)
        pend = {}
        for i in range(n_outer):
            b3 = _i32(i % 3)
            b2 = _i32(i % 2)
            for cp in idx_cps.pop(i):
                cp.wait()
            if i >= 2:
                pend.pop(i - 2).wait()
            if i + 1 < n_outer:
                fire_idx(i + 1)
            gcp = pltpu.async_copy(table.at[sidx.at[b3]], rows.at[b2], gsem)
            gcp.wait()
            pend[i] = pltpu.async_copy(rows.at[b2], accsh.at[segv.at[b3]],
                                       ssem, add=True)
        for gi in sorted(pend):
            pend[gi].wait()

    for k in range(stripe // 250):
        pltpu.sync_copy(zb, accsh.at[pl.ds(r0 + k * 250, 250)])
    plsc.subcore_barrier()
    for ch in range(2):
        @pl.when(core == ch)
        def _(ch=ch):
            scatter_pass(tables[ch])
    plsc.subcore_barrier()
    plane = sub // _i32(N // (SEGROWS // NS))
    prow = (sub % _i32(N // (SEGROWS // NS))) * _i32(stripe)
    for ch in range(2):
        @pl.when(core == ch)
        def _(ch=ch):
            pltpu.sync_copy(accsh.at[pl.ds(r0, stripe)],
                            outs[ch].at[plane, pl.ds(prow, stripe)])


@jax.jit
def _agg_kernel(t0, t1, src1, seg1):
    mesh = plsc.VectorSubcoreMesh(**_MESH)
    out = jax.ShapeDtypeStruct((R, N, CW), jnp.bfloat16)
    return pl.kernel(
        _agg_body,
        out_type=[out, out],
        mesh=mesh,
        compiler_params=pltpu.CompilerParams(use_tc_tiling_on_sc=False),
        scratch_types=[
            pltpu.VMEM_SHARED((SEGROWS, CW), jnp.bfloat16),
            pltpu.VMEM((3, STREAM), jnp.int32),
            pltpu.VMEM((3, STREAM), jnp.int32),
            pltpu.VMEM((2, STREAM, CW), jnp.bfloat16),
            pltpu.VMEM((250, CW), jnp.bfloat16),
            pltpu.SemaphoreType.DMA,
            pltpu.SemaphoreType.DMA,
            pltpu.SemaphoreType.DMA,
        ],
    )(t0, t1, src1, seg1)


BN = 200


def _combine_body_hc(a0, a1, a2, a3, cnt, xr, wfull, wroot, b,
                     h, hc0, hc1, hc2, hc3):
    _combine_core(a0, a1, a2, a3, cnt, xr, wfull, wroot, b,
                  h, [hc0, hc1, hc2, hc3])


def _combine_body_last(a0, a1, a2, a3, cnt, xr, wfull, wroot, b, h):
    _combine_core(a0, a1, a2, a3, cnt, xr, wfull, wroot, b, h, None)


def _combine_core(a0, a1, a2, a3, cnt, xr, wfull, wroot, b, h, hcs):
    cntarr = cnt[...]
    rec = 1.0 / jnp.maximum(cntarr[0] + cntarr[1], 1.0)
    acc = jnp.dot(xr[...], wroot[...], preferred_element_type=jnp.float32)
    arefs = [a0, a1, a2, a3]
    pieces = []
    for r in range(R):
        mr = jnp.concatenate([arefs[c][r] for c in range(NCH)], axis=1)
        pieces.append(mr.astype(jnp.float32) * rec[:, r:r + 1])
    m = jnp.concatenate(pieces, axis=1)
    acc = acc + jnp.dot(m, wfull[...], preferred_element_type=jnp.float32)
    out = jnp.maximum(acc + b[...], 0.0)
    h[...] = out
    if hcs is not None:
        for c in range(NCH):
            hcs[c][...] = out[:, c * CW:(c + 1) * CW].astype(jnp.bfloat16)


def _make_combine(emit_hc):
    body = _combine_body_hc if emit_hc else _combine_body_last
    z = lambda: jnp.int32(0)
    ablock = pl.BlockSpec((R, BN, CW), lambda i: (z(), i, z()))
    out_specs = [
        pl.BlockSpec((BN, D), lambda i: (i, z())),
    ]
    out_shape = [
        jax.ShapeDtypeStruct((N, D), jnp.float32),
    ]
    if emit_hc:
        out_specs += [pl.BlockSpec((BN, CW), lambda i: (i, z()))] * NCH
        out_shape += [jax.ShapeDtypeStruct((N, CW), jnp.bfloat16)] * NCH

    @jax.jit
    def run(a0, a1, a2, a3, cnt, xin, wfull, wroot, b):
        return pl.pallas_call(
            body,
            grid=(N // BN,),
            in_specs=[
                ablock, ablock, ablock, ablock,
                pl.BlockSpec((2, BN, R), lambda i: (z(), i, z())),
                pl.BlockSpec((BN, D), lambda i: (i, z())),
                pl.BlockSpec((R * D, D), lambda i: (z(), z())),
                pl.BlockSpec((D, D), lambda i: (z(), z())),
                pl.BlockSpec((1, D), lambda i: (z(), z())),
            ],
            out_specs=out_specs,
            out_shape=out_shape,
        )(a0, a1, a2, a3, cnt, xin, wfull, wroot, b)
    return run


_combine_hc = _make_combine(True)
_combine_last = _make_combine(False)


def _layer(tables, src1, seg1, cnt, xin, Wr, Wroot, b, last):
    a01 = _agg_kernel(tables[0], tables[1], src1, seg1)
    a23 = _agg_kernel(tables[2], tables[3], src1, seg1)
    a = list(a01) + list(a23)
    wfull = Wr.astype(jnp.float32).reshape(R * D, D)
    fn = _combine_last if last else _combine_hc
    res = fn(a[0], a[1], a[2], a[3], cnt, xin, wfull,
             Wroot.astype(jnp.float32),
             b.reshape(1, D).astype(jnp.float32))
    if last:
        return res[0], []
    return res[0], list(res[1:])


def kernel(x, edge_index, edge_attr, Wr1, Wroot1, b1, Wr2, Wroot2, b2):
    x = x.astype(jnp.float32)
    src1 = edge_index[0].astype(jnp.int32)
    dst1 = edge_index[1].astype(jnp.int32)
    et1 = edge_attr[:, 0].astype(jnp.int32)

    seg1, cnt_a, cnt_b = _count_kernel(dst1, et1)
    cnt = jnp.stack([cnt_a[:, 0].reshape(N, R), cnt_b[:, 0].reshape(N, R)])

    xb = x.astype(jnp.bfloat16)
    tables1 = [xb[:, c * CW:(c + 1) * CW] for c in range(NCH)]
    h1, tables2 = _layer(tables1, src1, seg1, cnt, x, Wr1, Wroot1, b1,
                         last=False)
    h2, _ = _layer(tables2, src1, seg1, cnt, h1, Wr2, Wroot2, b2,
                   last=True)
    return jnp.stack([h1, h2]).astype(jnp.float64)
